# baseline (speedup 1.0000x reference)
import jax
import jax.numpy as jnp
from jax import lax
from jax.experimental import pallas as pl
from jax.experimental.pallas import tpu as pltpu
from jax.experimental.pallas import tpu_sc as plsc

NC = 2
NS = 16
NW = NC * NS
L = 16

B = 16384
POS = 20
NEG = 200
K = POS + NEG
KP = 224
E = 64
CB = 2
NGH = 2
GH = NEG // NGH
CR = CB * K
BW = B // NW
CHUNKS = BW // CB
CIDX_GW = 128
CIDX_NG = BW // CIDX_GW
NGRP_R = 7


def _sc_body(cidx_hbm, pos_hbm, neg_hbm, inemb_hbm, outemb_hbm, loss_hbm,
             cidx_v, centers_v, idxp_a, idxp_b, idxn_a, idxn_b,
             rows_a, rows_b, stage_v, loss_v,
             sem_rows_a, sem_rows_b, sem_idx):
    wid = lax.axis_index("s") * NC + lax.axis_index("c")
    b0w = wid * BW

    pltpu.sync_copy(cidx_hbm.at[wid], cidx_v)
    for g in range(CIDX_NG):
        pltpu.async_copy(inemb_hbm.at[cidx_v.at[g]], centers_v.at[g],
                         sem_rows_a).wait()

    idxp_bufs = (idxp_a, idxp_b)
    idxn_bufs = (idxn_a, idxn_b)
    rows_bufs = (rows_a, rows_b)
    sem_rows = (sem_rows_a, sem_rows_b)
    lanes = lax.broadcasted_iota(jnp.int32, (L,), 0)
    mask15 = lanes == (L - 1)

    def idx_copies(b0, idxp_v, idxn_v, start):
        op = pltpu.async_copy if start else pltpu.make_async_copy
        return (op(pos_hbm.at[pl.ds(b0, CB)], idxp_v, sem_idx),
                op(neg_hbm.at[pl.ds(b0, CB)], idxn_v, sem_idx))

    def gathers(idxp_v, idxn_v, rows_v, sem, start):
        op = pltpu.async_copy if start else pltpu.make_async_copy
        cps = []
        for bl in range(CB):
            base = bl * K
            cps.append(op(outemb_hbm.at[idxp_v.at[bl]],
                          rows_v.at[pl.ds(base, POS)], sem))
            for g in range(NGH):
                cps.append(op(outemb_hbm.at[idxn_v.at[bl, g]],
                              rows_v.at[pl.ds(base + POS + g * GH, GH)], sem))
        return cps

    pltpu.sync_copy(pos_hbm.at[pl.ds(b0w, CB)], idxp_a)
    pltpu.sync_copy(neg_hbm.at[pl.ds(b0w, CB)], idxn_a)
    gathers(idxp_a, idxn_a, rows_a, sem_rows_a, start=True)
    idx_copies(b0w + CB, idxp_b, idxn_b, start=True)

    def outer_body(c2, carry):
        for sub in range(2):
            c = c2 * 2 + sub
            buf = sub
            nbuf = 1 - sub
            idxp_c, idxn_c, rows_c = idxp_bufs[buf], idxn_bufs[buf], rows_bufs[buf]
            idxp_n, idxn_n, rows_n = idxp_bufs[nbuf], idxn_bufs[nbuf], rows_bufs[nbuf]
            b0 = b0w + c * CB

            @pl.when(c + 1 < CHUNKS)
            def _():
                for cp in idx_copies(b0 + CB, idxp_n, idxn_n, start=False):
                    cp.wait()
                gathers(idxp_n, idxn_n, rows_n, sem_rows[nbuf], start=True)

            for cp in gathers(idxp_c, idxn_c, rows_c, sem_rows[buf],
                              start=False):
                cp.wait()

            @pl.when(c + 2 < CHUNKS)
            def _():
                idx_copies(b0 + 2 * CB, idxp_c, idxn_c, start=True)

            for b_local in range(CB):
                cb = c * CB + b_local
                chi = cb // CIDX_GW
                clo = cb % CIDX_GW
                fchi = jnp.broadcast_to(chi, (L,)).astype(jnp.int32)
                fclo = jnp.broadcast_to(clo, (L,)).astype(jnp.int32)
                cperm = [plsc.load_gather(centers_v, [fchi, fclo, pidx])
                         for pidx in (2 * lanes, 2 * lanes + 1,
                                      2 * lanes + 2 * L, 2 * lanes + 2 * L + 1)]

                def grp_body(jg, lacc, b_local=b_local, rows_c=rows_c,
                             cperm=cperm):
                    for u in range(L):
                        r = jnp.minimum(jg * L + u, K - 1)
                        base = b_local * K + r
                        h0 = rows_c[base, pl.ds(0, 2 * L)]
                        h1 = rows_c[base, pl.ds(2 * L, 2 * L)]
                        a0, b0 = plsc.unpack(
                            h0, format=plsc.PackFormat.INTERLEAVED,
                            preferred_element_type=jnp.float32)
                        a1, b1 = plsc.unpack(
                            h1, format=plsc.PackFormat.INTERLEAVED,
                            preferred_element_type=jnp.float32)
                        p = ((a0 * cperm[0] + b0 * cperm[1])
                             + (a1 * cperm[2] + b1 * cperm[3]))
                        stage_v[u, pl.ds(0, L)] = p
                    acc0 = jnp.zeros((L,), jnp.float32)
                    acc1 = jnp.zeros((L,), jnp.float32)
                    for col in range(L):
                        gth = plsc.load_gather(
                            stage_v, [lanes, jnp.full((L,), col, jnp.int32)])
                        if col % 2 == 0:
                            acc0 = acc0 + gth
                        else:
                            acc1 = acc1 + gth
                    dot = acc0 + acc1
                    jvec = jnp.broadcast_to(jg * L, (L,)).astype(jnp.int32) + lanes
                    x = jnp.where(jvec < POS, dot, -dot)
                    ea = jnp.exp(-jnp.abs(x))
                    s = ea / (2.0 + ea)
                    s2 = s * s
                    poly = 1.0 + s2 * ((1.0 / 3.0) + s2 * ((1.0 / 5.0)
                           + s2 * ((1.0 / 7.0) + s2 * (1.0 / 9.0))))
                    ls = jnp.minimum(x, 0.0) - 2.0 * s * poly
                    lacc = lacc + jnp.where(jvec < K, ls, 0.0)
                    return lacc

                lacc = lax.fori_loop(0, KP // L, grp_body,
                                     jnp.zeros((L,), jnp.float32))
                cs = plsc.cumsum(-lacc)
                plsc.store_scatter(loss_v, [jnp.full((L,), cb, jnp.int32)],
                                   cs, mask=mask15)
        return carry

    lax.fori_loop(0, CHUNKS // 2, outer_body, 0)

    pltpu.sync_copy(loss_v, loss_hbm.at[wid])


@jax.jit
def _sc_loss(cidx, pos3, neg3, in_embed, out_embed):
    mesh = plsc.VectorSubcoreMesh(core_axis_name="c", subcore_axis_name="s")
    return pl.kernel(
        _sc_body,
        out_type=jax.ShapeDtypeStruct((NW, BW), jnp.float32),
        mesh=mesh,
        scratch_types=[
            pltpu.VMEM((CIDX_NG, CIDX_GW), jnp.int32),
            pltpu.VMEM((CIDX_NG, CIDX_GW, E), jnp.float32),
            pltpu.VMEM((CB, POS), jnp.int32),
            pltpu.VMEM((CB, POS), jnp.int32),
            pltpu.VMEM((CB, NGH, GH), jnp.int32),
            pltpu.VMEM((CB, NGH, GH), jnp.int32),
            pltpu.VMEM((CR, E), jnp.bfloat16),
            pltpu.VMEM((CR, E), jnp.bfloat16),
            pltpu.VMEM((L, L + 1), jnp.float32),
            pltpu.VMEM((BW,), jnp.float32),
            pltpu.SemaphoreType.DMA,
            pltpu.SemaphoreType.DMA,
            pltpu.SemaphoreType.DMA,
        ],
        compiler_params=pltpu.CompilerParams(
            use_tc_tiling_on_sc=False, needs_layout_passes=False),
    )(cidx, pos3, neg3, in_embed, out_embed)


def kernel(input_labels, pos_labels, neg_labels, in_embed, out_embed):
    cidx = input_labels.astype(jnp.int32).reshape(NW, CIDX_NG, CIDX_GW)
    pos3 = pos_labels.astype(jnp.int32)
    neg3 = neg_labels.astype(jnp.int32).reshape(B, NGH, GH)
    out_bf = out_embed.astype(jnp.bfloat16)
    loss = _sc_loss(cidx, pos3, neg3, in_embed, out_bf)
    return loss.reshape(B)


# speedup vs baseline: 1.1471x; 1.1471x over previous
import jax
import jax.numpy as jnp
from jax import lax
from jax.experimental import pallas as pl
from jax.experimental.pallas import tpu as pltpu
from jax.experimental.pallas import tpu_sc as plsc

NC = 2
NS = 16
NW = NC * NS
L = 16

B = 16384
POS = 20
NEG = 200
K = POS + NEG
KP = 224
E = 64
CB = 2
NGH = 2
GH = NEG // NGH
CR = CB * K
BW = B // NW
CHUNKS = BW // CB
NGRP_R = 7


def _sc_body(centers_hbm, pos_hbm, neg_hbm, outemb_hbm, loss_hbm,
             centers_v, idxp_a, idxp_b, idxn_a, idxn_b,
             rows_a, rows_b, loss_v,
             sem_rows_a, sem_rows_b, sem_idx):
    wid = lax.axis_index("s") * NC + lax.axis_index("c")
    b0w = wid * BW

    pltpu.sync_copy(centers_hbm.at[pl.ds(b0w, BW)], centers_v)

    idxp_bufs = (idxp_a, idxp_b)
    idxn_bufs = (idxn_a, idxn_b)
    rows_bufs = (rows_a, rows_b)
    sem_rows = (sem_rows_a, sem_rows_b)
    lanes = lax.broadcasted_iota(jnp.int32, (L,), 0)
    mask15 = lanes == (L - 1)

    def idx_copies(b0, idxp_v, idxn_v, start):
        op = pltpu.async_copy if start else pltpu.make_async_copy
        return (op(pos_hbm.at[pl.ds(b0, CB)], idxp_v, sem_idx),
                op(neg_hbm.at[pl.ds(b0, CB)], idxn_v, sem_idx))

    def gathers(idxp_v, idxn_v, rows_v, sem, start):
        op = pltpu.async_copy if start else pltpu.make_async_copy
        cps = []
        for bl in range(CB):
            base = bl * K
            cps.append(op(outemb_hbm.at[idxp_v.at[bl]],
                          rows_v.at[pl.ds(base, POS)], sem))
            for g in range(NGH):
                cps.append(op(outemb_hbm.at[idxn_v.at[bl, g]],
                              rows_v.at[pl.ds(base + POS + g * GH, GH)], sem))
        return cps

    pltpu.sync_copy(pos_hbm.at[pl.ds(b0w, CB)], idxp_a)
    pltpu.sync_copy(neg_hbm.at[pl.ds(b0w, CB)], idxn_a)
    gathers(idxp_a, idxn_a, rows_a, sem_rows_a, start=True)
    idx_copies(b0w + CB, idxp_b, idxn_b, start=True)

    def outer_body(c2, carry):
        for sub in range(2):
            c = c2 * 2 + sub
            buf = sub
            nbuf = 1 - sub
            idxp_c, idxn_c, rows_c = idxp_bufs[buf], idxn_bufs[buf], rows_bufs[buf]
            idxp_n, idxn_n, rows_n = idxp_bufs[nbuf], idxn_bufs[nbuf], rows_bufs[nbuf]
            b0 = b0w + c * CB

            @pl.when(c + 1 < CHUNKS)
            def _():
                for cp in idx_copies(b0 + CB, idxp_n, idxn_n, start=False):
                    cp.wait()
                gathers(idxp_n, idxn_n, rows_n, sem_rows[nbuf], start=True)

            for cp in gathers(idxp_c, idxn_c, rows_c, sem_rows[buf],
                              start=False):
                cp.wait()

            @pl.when(c + 2 < CHUNKS)
            def _():
                idx_copies(b0 + 2 * CB, idxp_c, idxn_c, start=True)

            for b_local in range(CB):
                cb = c * CB + b_local
                fcb = jnp.broadcast_to(cb, (L,)).astype(jnp.int32)

                def rnd_body(rnd, lacc, b_local=b_local, rows_c=rows_c,
                             fcb=fcb):
                    bK = b_local * K
                    base0 = bK + rnd * (NGRP_R * L)
                    i0s = [jnp.minimum(base0 + g * L + lanes, bK + K - 1)
                           for g in range(NGRP_R)]
                    accs = [jnp.zeros((L,), jnp.float32)
                            for _ in range(NGRP_R)]
                    for cc in range(E):
                        rot = (lanes + cc) & (E - 1)
                        cvr = plsc.load_gather(centers_v, [fcb, rot])
                        for g in range(NGRP_R):
                            gth = plsc.load_gather(rows_c, [i0s[g], rot])
                            accs[g] = accs[g] + gth * cvr
                    for g in range(NGRP_R):
                        jvec = jnp.broadcast_to(
                            rnd * (NGRP_R * L) + g * L, (L,)
                        ).astype(jnp.int32) + lanes
                        x = jnp.where(jvec < POS, accs[g], -accs[g])
                        ea = jnp.exp(-jnp.abs(x))
                        s = ea / (2.0 + ea)
                        s2 = s * s
                        poly = 1.0 + s2 * ((1.0 / 3.0) + s2 * ((1.0 / 5.0)
                               + s2 * ((1.0 / 7.0) + s2 * (1.0 / 9.0))))
                        ls = jnp.minimum(x, 0.0) - 2.0 * s * poly
                        lacc = lacc + jnp.where(jvec < K, ls, 0.0)
                    return lacc

                lacc = lax.fori_loop(0, KP // (NGRP_R * L), rnd_body,
                                     jnp.zeros((L,), jnp.float32))
                cs = plsc.cumsum(-lacc)
                plsc.store_scatter(loss_v, [jnp.full((L,), cb, jnp.int32)],
                                   cs, mask=mask15)
        return carry

    lax.fori_loop(0, CHUNKS // 2, outer_body, 0)

    pltpu.sync_copy(loss_v, loss_hbm.at[wid])


@jax.jit
def _sc_loss(centers, pos3, neg3, out_embed):
    mesh = plsc.VectorSubcoreMesh(core_axis_name="c", subcore_axis_name="s")
    return pl.kernel(
        _sc_body,
        out_type=jax.ShapeDtypeStruct((NW, BW), jnp.float32),
        mesh=mesh,
        scratch_types=[
            pltpu.VMEM((BW, E), jnp.float32),
            pltpu.VMEM((CB, POS), jnp.int32),
            pltpu.VMEM((CB, POS), jnp.int32),
            pltpu.VMEM((CB, NGH, GH), jnp.int32),
            pltpu.VMEM((CB, NGH, GH), jnp.int32),
            pltpu.VMEM((CR, E), jnp.float32),
            pltpu.VMEM((CR, E), jnp.float32),
            pltpu.VMEM((BW,), jnp.float32),
            pltpu.SemaphoreType.DMA,
            pltpu.SemaphoreType.DMA,
            pltpu.SemaphoreType.DMA,
        ],
        compiler_params=pltpu.CompilerParams(
            use_tc_tiling_on_sc=False, needs_layout_passes=False),
    )(centers, pos3, neg3, out_embed)


def kernel(input_labels, pos_labels, neg_labels, in_embed, out_embed):
    centers = jnp.take(in_embed, input_labels.astype(jnp.int32).reshape(B),
                       axis=0)
    pos3 = pos_labels.astype(jnp.int32)
    neg3 = neg_labels.astype(jnp.int32).reshape(B, NGH, GH)
    loss = _sc_loss(centers, pos3, neg3, out_embed)
    return loss.reshape(B)
